# Initial kernel scaffold; baseline (speedup 1.0000x reference)
import jax
import jax.numpy as jnp
from jax.experimental import pallas as pl


def kernel(x, w1, b1, w2, b2, w3, b3, wf1, bf1, wf2, bf2):
    raise NotImplementedError("write your pallas kernel here")


# speedup vs baseline: 2.3653x; 2.3653x over previous
import jax
import jax.numpy as jnp
from jax.experimental import pallas as pl
from jax.experimental.pallas import tpu as pltpu

f32 = jnp.float32
bf16 = jnp.bfloat16

H0 = 80
H1, P1 = 76, 37
H2, P2 = 35, 17
H3, P3 = 16, 7
C1, C2, C3 = 32, 32, 64
FC1_OUT = 40
OUT_LANES = 128

X2_ROWS, X2_ALLOC = P1 * P1, P1 * P1 + 7
X3_ROWS, X3_ALLOC = P2 * P2, P2 * P2 + 7
C2_ROWS = H2 * P1
C3_ROWS = H3 * P2
C2_CHUNK = 5 * P1
C3_CHUNK = 8 * P2
BLK = 8
NBLK = 10
KROWS = BLK + 4
FEAT_ROWS = 56


def _feat_kernel(x_ref, B1_ref, b1_ref, W2_ref, b2_ref, W3_ref, b3_ref,
                 o_ref, xsh_ref, lhsT_ref, o1_ref, hp_ref, x2_ref, p2_ref,
                 c2_ref, x3_ref, p3_ref, c3_ref):
    for kw in range(5):
        xsh_ref[kw, 0:H0, :] = x_ref[:, kw:kw + H1].astype(bf16)
    xsh_ref[:, H0:H0 + 8, :] = jnp.zeros((5, 8, H1), bf16)

    for b in range(NBLK):
        h0 = b * BLK
        for kw in range(5):
            lhsT_ref[pl.ds(kw * KROWS, KROWS), :] = xsh_ref[kw, pl.ds(h0, KROWS), :]
        blk = jax.lax.dot_general(
            lhsT_ref[...], B1_ref[...],
            (((0,), (0,)), ((), ())), preferred_element_type=f32)
        o1_ref[b] = blk.astype(bf16)

    for i in range(P1):
        parts = []
        for k in range(3):
            h = 2 * i + k
            parts.append(o1_ref[h // BLK, :, (h % BLK) * C1:(h % BLK) * C1 + C1])
        hp_ref[...] = jnp.maximum(jnp.maximum(parts[0], parts[1]),
                                  parts[2]).astype(f32)
        wp = jnp.maximum(
            jnp.maximum(hp_ref[pl.ds(0, P1, stride=2), :],
                        hp_ref[pl.ds(1, P1, stride=2), :]),
            hp_ref[pl.ds(2, P1, stride=2), :])
        p = jnp.maximum(wp + b1_ref[...], 0.0)
        x2_ref[pl.ds(i * P1, P1), :] = p.astype(bf16)
    x2_ref[pl.ds(X2_ROWS, X2_ALLOC - X2_ROWS), :] = jnp.zeros(
        (X2_ALLOC - X2_ROWS, C1), bf16)

    for c in range(H2 // 5):
        base = c * C2_CHUNK
        for kh in range(3):
            for kwi in range(3):
                j = kh * 3 + kwi
                p2_ref[:, j * C1:(j + 1) * C1] = x2_ref[
                    pl.ds(base + kh * P1 + kwi, C2_CHUNK), :]
        acc = jnp.dot(p2_ref[...], W2_ref[...], preferred_element_type=f32)
        c2_ref[pl.ds(base, C2_CHUNK), :] = acc

    for i in range(P2):
        r = 2 * i
        m = c2_ref[pl.ds(r * P1, P2, stride=2), :]
        for k in range(3):
            for l in range(3):
                if k == 0 and l == 0:
                    continue
                m = jnp.maximum(
                    m, c2_ref[pl.ds((r + k) * P1 + l, P2, stride=2), :])
        p = jnp.maximum(m + b2_ref[...], 0.0)
        x3_ref[pl.ds(i * P2, P2), :] = p.astype(bf16)
    x3_ref[pl.ds(X3_ROWS, X3_ALLOC - X3_ROWS), :] = jnp.zeros(
        (X3_ALLOC - X3_ROWS, C2), bf16)

    for c in range(C3_ROWS // C3_CHUNK):
        base = c * C3_CHUNK
        for kh in range(2):
            for kwi in range(2):
                j = kh * 2 + kwi
                p3_ref[:, j * C2:(j + 1) * C2] = x3_ref[
                    pl.ds(base + kh * P2 + kwi, C3_CHUNK), :]
        acc = jnp.dot(p3_ref[...], W3_ref[...], preferred_element_type=f32)
        c3_ref[pl.ds(base, C3_CHUNK), :] = acc

    for i in range(P3):
        r = 2 * i
        m = c3_ref[pl.ds(r * P2, P3, stride=2), :]
        for k in range(3):
            for l in range(3):
                if k == 0 and l == 0:
                    continue
                m = jnp.maximum(
                    m, c3_ref[pl.ds((r + k) * P2 + l, P3, stride=2), :])
        o_ref[pl.ds(i * P3, P3), :] = jnp.maximum(m + b3_ref[...], 0.0)
    o_ref[pl.ds(P3 * P3, FEAT_ROWS - P3 * P3), :] = jnp.zeros(
        (FEAT_ROWS - P3 * P3, C3), f32)


def _features(x_img, B1, b1, W2, b2, W3, b3):
    n = x_img.shape[0]
    return pl.pallas_call(
        _feat_kernel,
        out_shape=jax.ShapeDtypeStruct((n, FEAT_ROWS, C3), f32),
        grid=(n,),
        in_specs=[
            pl.BlockSpec((None, H0, H0), lambda i: (i, 0, 0)),
            pl.BlockSpec((5 * KROWS, BLK * C1), lambda i: (0, 0)),
            pl.BlockSpec((1, C1), lambda i: (0, 0)),
            pl.BlockSpec((9 * C1, C2), lambda i: (0, 0)),
            pl.BlockSpec((1, C2), lambda i: (0, 0)),
            pl.BlockSpec((4 * C2, C3), lambda i: (0, 0)),
            pl.BlockSpec((1, C3), lambda i: (0, 0)),
        ],
        out_specs=pl.BlockSpec((None, FEAT_ROWS, C3), lambda i: (i, 0, 0)),
        scratch_shapes=[
            pltpu.VMEM((5, H0 + 8, H1), bf16),
            pltpu.VMEM((5 * KROWS, H1), bf16),
            pltpu.VMEM((NBLK, H1, BLK * C1), bf16),
            pltpu.VMEM((H1, C1), f32),
            pltpu.VMEM((X2_ALLOC, C1), bf16),
            pltpu.VMEM((C2_CHUNK, 9 * C1), bf16),
            pltpu.VMEM((C2_ROWS + 1, C2), f32),
            pltpu.VMEM((X3_ALLOC, C2), bf16),
            pltpu.VMEM((C3_CHUNK, 4 * C2), bf16),
            pltpu.VMEM((C3_ROWS, C3), f32),
        ],
        compiler_params=pltpu.CompilerParams(
            dimension_semantics=("parallel",),
        ),
    )(x_img, B1, b1, W2, b2, W3, b3)


def _head_kernel(f_ref, wf1_ref, bf1_ref, wf2_ref, bf2_ref, o_ref):
    h = jnp.dot(f_ref[...], wf1_ref[...], preferred_element_type=f32)
    h = jnp.maximum(h + bf1_ref[...], 0.0)
    o_ref[...] = jnp.dot(h, wf2_ref[...],
                         preferred_element_type=f32) + bf2_ref[...]


def _head(feats, wf1r, bf1, wf2p, bf2p):
    n, d = feats.shape
    tile = 128
    return pl.pallas_call(
        _head_kernel,
        out_shape=jax.ShapeDtypeStruct((n, OUT_LANES), f32),
        grid=(n // tile,),
        in_specs=[
            pl.BlockSpec((tile, d), lambda i: (i, 0)),
            pl.BlockSpec((d, FC1_OUT), lambda i: (0, 0)),
            pl.BlockSpec((1, FC1_OUT), lambda i: (0, 0)),
            pl.BlockSpec((FC1_OUT, OUT_LANES), lambda i: (0, 0)),
            pl.BlockSpec((1, OUT_LANES), lambda i: (0, 0)),
        ],
        out_specs=pl.BlockSpec((tile, OUT_LANES), lambda i: (i, 0)),
        compiler_params=pltpu.CompilerParams(
            dimension_semantics=("parallel",),
        ),
    )(feats, wf1r, bf1, wf2p, bf2p)


@jax.jit
def kernel(x, w1, b1, w2, b2, w3, b3, wf1, bf1, wf2, bf2):
    n = x.shape[0]
    n_act = wf2.shape[1]
    x_img = x.reshape(n, H0, H0)

    B = jnp.zeros((5, KROWS, BLK, C1), f32)
    for dr in range(BLK):
        for kh in range(5):
            B = B.at[:, dr + kh, dr, :].set(w1[kh, :, 0, :])
    B1 = B.reshape(5 * KROWS, BLK * C1).astype(bf16)

    W2 = w2.reshape(9 * C1, C2).astype(bf16)
    W3 = w3.reshape(4 * C2, C3).astype(bf16)

    feats = _features(x_img, B1, b1, W2, b2, W3, b3)
    f = feats[:, :P3 * P3, :].reshape(n, P3 * P3 * C3)

    wf1r = wf1.reshape(C3, P3 * P3, FC1_OUT).transpose(1, 0, 2).reshape(
        P3 * P3 * C3, FC1_OUT)
    wf2p = jnp.zeros((FC1_OUT, OUT_LANES), f32).at[:, :n_act].set(wf2)
    bf2p = jnp.zeros((1, OUT_LANES), f32).at[:, :n_act].set(bf2)

    out = _head(f, wf1r, bf1, wf2p, bf2p)
    return out[:, :n_act]


# speedup vs baseline: 2.7358x; 1.1567x over previous
import jax
import jax.numpy as jnp
from jax.experimental import pallas as pl
from jax.experimental.pallas import tpu as pltpu

f32 = jnp.float32
bf16 = jnp.bfloat16

H0 = 80
H1, P1 = 76, 37
H2, P2 = 35, 17
H3, P3 = 16, 7
C1, C2, C3 = 32, 32, 64
FC1_OUT = 40
OUT_LANES = 128

BLK = 8
NBLK = 10
KROWS = 12
KPAD = 16

PITCH1 = 40
PITCH2 = 24
FPITCH = 8
X2_ALLOC = 1488
X3_ALLOC = 416
C2_CHUNK = 5 * PITCH1
C3_CHUNK = 8 * PITCH2
C2_ALLOC = 7 * C2_CHUNK + 8
C3_ALLOC = 2 * C3_CHUNK
FEAT_ROWS = P3 * FPITCH


def _feat_kernel(x_ref, B1_ref, b1_ref, W2_ref, b2_ref, W3_ref, b3_ref,
                 o_ref, xsh_ref, lhsT_ref, o1_ref, hp_ref, x2_ref, y2_ref,
                 c2_ref, x3_ref, y3_ref, c3_ref):
    for kw in range(5):
        xsh_ref[kw, 0:H0, :] = x_ref[:, kw:kw + H1].astype(bf16)
    xsh_ref[:, H0:H0 + 8, :] = jnp.zeros((5, 8, H1), bf16)
    lhsT_ref[...] = jnp.zeros((5, KPAD, H1), bf16)

    for b in range(NBLK):
        h0 = b * BLK
        lhsT_ref[:, 0:KROWS, :] = xsh_ref[:, pl.ds(h0, KROWS), :]
        blk = jax.lax.dot_general(
            lhsT_ref[...].reshape(5 * KPAD, H1), B1_ref[...],
            (((0,), (0,)), ((), ())), preferred_element_type=f32)
        o1_ref[b] = blk.astype(bf16)

    x2_ref[...] = jnp.zeros((X2_ALLOC, C1), bf16)
    for i in range(P1):
        parts = []
        for k in range(3):
            h = 2 * i + k
            parts.append(o1_ref[h // BLK, :, (h % BLK) * C1:(h % BLK) * C1 + C1])
        hp_ref[...] = jnp.maximum(jnp.maximum(parts[0], parts[1]),
                                  parts[2]).astype(f32)
        wp = jnp.maximum(
            jnp.maximum(hp_ref[pl.ds(0, P1, stride=2), :],
                        hp_ref[pl.ds(1, P1, stride=2), :]),
            hp_ref[pl.ds(2, P1, stride=2), :])
        p = jnp.maximum(wp + b1_ref[...], 0.0)
        x2_ref[pl.ds(i * PITCH1, P1), :] = p.astype(bf16)

    for c in range(12):
        base = c * 124
        y2_ref[pl.ds(base, 124), :] = jnp.dot(
            x2_ref[pl.ds(base, 124), :], W2_ref[...],
            preferred_element_type=f32).astype(bf16)
    for rb in (0, 700):
        acc = jnp.zeros((700, C2), f32)
        for kh in range(3):
            for kwi in range(3):
                j = kh * 3 + kwi
                acc = acc + y2_ref[
                    pl.ds(rb + kh * PITCH1 + kwi, 700),
                    j * C2:(j + 1) * C2].astype(f32)
        c2_ref[pl.ds(rb, 700), :] = acc

    x3_ref[...] = jnp.zeros((X3_ALLOC, C2), bf16)
    for i in range(P2):
        r = 2 * i
        m = c2_ref[pl.ds(r * PITCH1, P2, stride=2), :]
        for k in range(3):
            for l in range(3):
                if k == 0 and l == 0:
                    continue
                m = jnp.maximum(
                    m, c2_ref[pl.ds((r + k) * PITCH1 + l, P2, stride=2), :])
        p = jnp.maximum(m + b2_ref[...], 0.0)
        x3_ref[pl.ds(i * PITCH2, P2), :] = p.astype(bf16)

    for c in range(4):
        base = c * 104
        y3_ref[pl.ds(base, 104), :] = jnp.dot(
            x3_ref[pl.ds(base, 104), :], W3_ref[...],
            preferred_element_type=f32).astype(bf16)
    acc3 = jnp.zeros((C3_ALLOC, C3), f32)
    for kh in range(2):
        for kwi in range(2):
            j = kh * 2 + kwi
            acc3 = acc3 + y3_ref[
                pl.ds(kh * PITCH2 + kwi, C3_ALLOC),
                j * C3:(j + 1) * C3].astype(f32)
    c3_ref[...] = acc3

    o_ref[...] = jnp.zeros((FEAT_ROWS, C3), f32)
    for i in range(P3):
        r = 2 * i
        m = c3_ref[pl.ds(r * PITCH2, P3, stride=2), :]
        for k in range(3):
            for l in range(3):
                if k == 0 and l == 0:
                    continue
                m = jnp.maximum(
                    m, c3_ref[pl.ds((r + k) * PITCH2 + l, P3, stride=2), :])
        o_ref[pl.ds(i * FPITCH, P3), :] = jnp.maximum(m + b3_ref[...], 0.0)


def _features(x_img, B1, b1, W2, b2, W3, b3):
    n = x_img.shape[0]
    return pl.pallas_call(
        _feat_kernel,
        out_shape=jax.ShapeDtypeStruct((n, FEAT_ROWS, C3), f32),
        grid=(n,),
        in_specs=[
            pl.BlockSpec((None, H0, H0), lambda i: (i, 0, 0)),
            pl.BlockSpec((5 * KPAD, BLK * C1), lambda i: (0, 0)),
            pl.BlockSpec((1, C1), lambda i: (0, 0)),
            pl.BlockSpec((C1, 9 * C2), lambda i: (0, 0)),
            pl.BlockSpec((1, C2), lambda i: (0, 0)),
            pl.BlockSpec((C2, 4 * C3), lambda i: (0, 0)),
            pl.BlockSpec((1, C3), lambda i: (0, 0)),
        ],
        out_specs=pl.BlockSpec((None, FEAT_ROWS, C3), lambda i: (i, 0, 0)),
        scratch_shapes=[
            pltpu.VMEM((5, H0 + 8, H1), bf16),
            pltpu.VMEM((5, KPAD, H1), bf16),
            pltpu.VMEM((NBLK, H1, BLK * C1), bf16),
            pltpu.VMEM((H1, C1), f32),
            pltpu.VMEM((X2_ALLOC, C1), bf16),
            pltpu.VMEM((X2_ALLOC, 9 * C2), bf16),
            pltpu.VMEM((C2_ALLOC, C2), f32),
            pltpu.VMEM((X3_ALLOC, C2), bf16),
            pltpu.VMEM((X3_ALLOC, 4 * C3), bf16),
            pltpu.VMEM((C3_ALLOC, C3), f32),
        ],
        compiler_params=pltpu.CompilerParams(
            dimension_semantics=("parallel",),
        ),
    )(x_img, B1, b1, W2, b2, W3, b3)


def _head_kernel(f_ref, wf1_ref, bf1_ref, wf2_ref, bf2_ref, o_ref):
    h = jnp.dot(f_ref[...], wf1_ref[...], preferred_element_type=f32)
    h = jnp.maximum(h + bf1_ref[...], 0.0)
    o_ref[...] = jnp.dot(h, wf2_ref[...],
                         preferred_element_type=f32) + bf2_ref[...]


def _head(feats, wf1r, bf1, wf2p, bf2p):
    n, d = feats.shape
    tile = 128
    return pl.pallas_call(
        _head_kernel,
        out_shape=jax.ShapeDtypeStruct((n, OUT_LANES), f32),
        grid=(n // tile,),
        in_specs=[
            pl.BlockSpec((tile, d), lambda i: (i, 0)),
            pl.BlockSpec((d, FC1_OUT), lambda i: (0, 0)),
            pl.BlockSpec((1, FC1_OUT), lambda i: (0, 0)),
            pl.BlockSpec((FC1_OUT, OUT_LANES), lambda i: (0, 0)),
            pl.BlockSpec((1, OUT_LANES), lambda i: (0, 0)),
        ],
        out_specs=pl.BlockSpec((tile, OUT_LANES), lambda i: (i, 0)),
        compiler_params=pltpu.CompilerParams(
            dimension_semantics=("parallel",),
        ),
    )(feats, wf1r, bf1, wf2p, bf2p)


@jax.jit
def kernel(x, w1, b1, w2, b2, w3, b3, wf1, bf1, wf2, bf2):
    n = x.shape[0]
    n_act = wf2.shape[1]
    x_img = x.reshape(n, H0, H0)

    B = jnp.zeros((5, KPAD, BLK, C1), f32)
    for dr in range(BLK):
        for kh in range(5):
            B = B.at[:, dr + kh, dr, :].set(w1[kh, :, 0, :])
    B1 = B.reshape(5 * KPAD, BLK * C1).astype(bf16)

    W2 = w2.transpose(2, 0, 1, 3).reshape(C1, 9 * C2).astype(bf16)
    W3 = w3.transpose(2, 0, 1, 3).reshape(C2, 4 * C3).astype(bf16)

    feats = _features(x_img, B1, b1, W2, b2, W3, b3)
    f = feats.reshape(n, P3, FPITCH, C3)[:, :, :P3, :].reshape(
        n, P3 * P3 * C3)

    wf1r = wf1.reshape(C3, P3 * P3, FC1_OUT).transpose(1, 0, 2).reshape(
        P3 * P3 * C3, FC1_OUT)
    wf2p = jnp.zeros((FC1_OUT, OUT_LANES), f32).at[:, :n_act].set(wf2)
    bf2p = jnp.zeros((1, OUT_LANES), f32).at[:, :n_act].set(bf2)

    out = _head(f, wf1r, bf1, wf2p, bf2p)
    return out[:, :n_act]
